# baseline (speedup 1.0000x reference)
import functools

import jax
import jax.numpy as jnp
from jax import lax
from jax.experimental import pallas as pl
from jax.experimental.pallas import tpu as pltpu
from jax.experimental.pallas import tpu_sc as plsc

N = 10000
E = 640000
C = 128
XW = 16
CW = 64
KP = 1

NC = 2
NS = 16
NW = NC * NS
CHUNK = 128
N_PAD = 10112
ROWS_PT = N_PAD // NS
CH32 = 160
CH16 = 320
E_PAD = NW * CH32 * CHUNK

PIPE = 5


def _gather_scatter_loop(table, ed_hbm, base, ed_v, rows, acc,
                         isem, gsem, ssem, n_chunks):
    def round_body(r, first):
        for b in range(PIPE):
            j = base + r * PIPE + b
            if not first:
                pltpu.make_async_copy(rows.at[b], acc.at[ed_v.at[b, 1]],
                                      ssem.at[b]).wait()
            pltpu.async_copy(ed_hbm.at[j], ed_v.at[b], isem.at[b])
        for b in range(PIPE):
            j = base + r * PIPE + b
            pltpu.make_async_copy(ed_hbm.at[j], ed_v.at[b], isem.at[b]).wait()
            pltpu.async_copy(table.at[ed_v.at[b, 0]], rows.at[b], gsem.at[b])
        for b in range(PIPE):
            pltpu.make_async_copy(table.at[ed_v.at[b, 0]], rows.at[b],
                                  gsem.at[b]).wait()
            pltpu.async_copy(rows.at[b], acc.at[ed_v.at[b, 1]], ssem.at[b],
                             add=True)

    round_body(0, True)

    def body(r, _):
        round_body(r, False)
        return 0

    lax.fori_loop(1, n_chunks // PIPE, body, 0, unroll=False)
    for b in range(PIPE):
        pltpu.make_async_copy(rows.at[b], acc.at[ed_v.at[b, 1]],
                              ssem.at[b]).wait()


_MESH = plsc.VectorSubcoreMesh(core_axis_name="c", subcore_axis_name="s")


@functools.partial(
    pl.kernel,
    out_type=jax.ShapeDtypeStruct((NC, N_PAD, XW), jnp.float32),
    mesh=_MESH,
    compiler_params=pltpu.CompilerParams(use_tc_tiling_on_sc=False),
    scratch_types=[
        pltpu.VMEM((PIPE, 2, CHUNK), jnp.int32),
        pltpu.VMEM((PIPE, CHUNK, XW), jnp.float32),
        pltpu.VMEM_SHARED((N_PAD, XW), jnp.float32),
        pltpu.SemaphoreType.DMA((PIPE,)),
        pltpu.SemaphoreType.DMA((PIPE,)),
        pltpu.SemaphoreType.DMA((PIPE,)),
    ],
)
def _segsum16(x_hbm, ed_hbm, zeros_hbm, out_hbm,
              ed_v, rows, acc, isem, gsem, ssem):
    cid = lax.axis_index("c")
    sid = lax.axis_index("s")
    wid = cid * NS + sid
    row0 = sid * ROWS_PT
    pltpu.sync_copy(zeros_hbm.at[pl.ds(row0, ROWS_PT)],
                    acc.at[pl.ds(row0, ROWS_PT)])
    plsc.subcore_barrier()
    _gather_scatter_loop(x_hbm, ed_hbm, wid * CH32, ed_v, rows, acc,
                         isem, gsem, ssem, CH32)
    plsc.subcore_barrier()
    pltpu.sync_copy(acc.at[pl.ds(row0, ROWS_PT)],
                    out_hbm.at[cid, pl.ds(row0, ROWS_PT)])


@functools.partial(
    pl.kernel,
    out_type=jax.ShapeDtypeStruct((NC, KP, N_PAD, CW), jnp.float32),
    mesh=_MESH,
    compiler_params=pltpu.CompilerParams(use_tc_tiling_on_sc=False),
    scratch_types=[
        pltpu.VMEM((PIPE, 2, CHUNK), jnp.int32),
        pltpu.VMEM((PIPE, CHUNK, CW), jnp.float32),
        pltpu.VMEM_SHARED((N_PAD, CW), jnp.float32),
        pltpu.SemaphoreType.DMA((PIPE,)),
        pltpu.SemaphoreType.DMA((PIPE,)),
        pltpu.SemaphoreType.DMA((PIPE,)),
    ],
)
def _segsum128(h_hbm, ed_hbm, zeros_hbm, out_hbm,
               ed_v, rows, acc, isem, gsem, ssem):
    cid = lax.axis_index("c")
    sid = lax.axis_index("s")
    row0 = sid * ROWS_PT
    for k in range(KP):
        pltpu.sync_copy(zeros_hbm.at[pl.ds(row0, ROWS_PT)],
                        acc.at[pl.ds(row0, ROWS_PT)])
        plsc.subcore_barrier()
        _gather_scatter_loop(h_hbm.at[cid, k], ed_hbm, sid * CH16, ed_v,
                             rows, acc, isem, gsem, ssem, CH16)
        plsc.subcore_barrier()
        pltpu.sync_copy(acc.at[pl.ds(row0, ROWS_PT)],
                        out_hbm.at[cid, k, pl.ds(row0, ROWS_PT)])


def _tc_body(split_out, p_ref, agg1_ref, h_ref, wl_ref, bl_ref, wr_ref, o_ref):
    if p_ref.ndim == 3:
        agg = p_ref[0] + p_ref[1]
    else:
        agg = jnp.concatenate(
            [p_ref[c, k] for c in range(NC) for k in range(KP)], axis=1)
    a1 = agg1_ref[0] + agg1_ref[1]
    deg = a1[:, 3:4]
    invd = 1.0 / jnp.maximum(deg, 1.0)
    mean = agg * invd
    h = h_ref[...]
    if h.ndim == 4:
        h = jnp.concatenate(
            [h[c, k] for c in range(NC) for k in range(KP)], axis=1)
    y = (jnp.dot(mean, wl_ref[...], preferred_element_type=jnp.float32)
         + bl_ref[...]
         + jnp.dot(h, wr_ref[...], preferred_element_type=jnp.float32))
    y = jnp.maximum(y, 0.0)
    if split_out:
        for c in range(NC):
            for k in range(KP):
                b = (c * KP + k) * CW
                o_ref[c, k] = y[:, b:b + CW]
    else:
        o_ref[...] = y


def _tc_layer(P, agg1, h, WlT, bl, WrT, split_out):
    BR = 1264
    grid = N_PAD // BR
    d_agg = WlT.shape[0]
    d_in = WrT.shape[0]
    p_spec = (pl.BlockSpec((NC, BR, XW), lambda i: (0, i, 0)) if P.ndim == 3
              else pl.BlockSpec((NC, KP, BR, CW), lambda i: (0, 0, i, 0)))
    h_spec = (pl.BlockSpec((NC, KP, BR, CW), lambda i: (0, 0, i, 0))
              if h.ndim == 4
              else pl.BlockSpec((BR, h.shape[1]), lambda i: (i, 0)))
    if split_out:
        out_spec = pl.BlockSpec((NC, KP, BR, CW), lambda i: (0, 0, i, 0))
        out_shape = jax.ShapeDtypeStruct((NC, KP, N_PAD, CW), jnp.float32)
    else:
        out_spec = pl.BlockSpec((BR, C), lambda i: (i, 0))
        out_shape = jax.ShapeDtypeStruct((N_PAD, C), jnp.float32)
    return pl.pallas_call(
        functools.partial(_tc_body, split_out),
        grid=(grid,),
        in_specs=[
            p_spec,
            pl.BlockSpec((NC, BR, XW), lambda i: (0, i, 0)),
            h_spec,
            pl.BlockSpec((d_agg, C), lambda i: (0, 0)),
            pl.BlockSpec((1, C), lambda i: (0, 0)),
            pl.BlockSpec((d_in, C), lambda i: (0, 0)),
        ],
        out_specs=out_spec,
        out_shape=out_shape,
    )(P, agg1, h, WlT, bl, WrT)


def kernel(x, edge_index, Wl1, bl1, Wr1, Wl2, bl2, Wr2, Wl3, bl3, Wr3):
    src = edge_index[0].astype(jnp.int32)
    dst = edge_index[1].astype(jnp.int32)
    src_p = jnp.concatenate([src, jnp.zeros((E_PAD - E,), jnp.int32)])
    dst_p = jnp.concatenate([dst, jnp.full((E_PAD - E,), N, jnp.int32)])
    ed = jnp.stack([src_p.reshape(E_PAD // CHUNK, CHUNK),
                    dst_p.reshape(E_PAD // CHUNK, CHUNK)], axis=1)

    x16 = jnp.zeros((N_PAD, XW), jnp.float32)
    x16 = x16.at[:N, :3].set(x).at[:N, 3].set(1.0)
    z16 = jnp.zeros((N_PAD, XW), jnp.float32)
    z64 = jnp.zeros((N_PAD, CW), jnp.float32)

    Wl1T = jnp.zeros((XW, C), jnp.float32).at[:3].set(Wl1.T)
    Wr1T = jnp.zeros((XW, C), jnp.float32).at[:3].set(Wr1.T)
    b1 = bl1.reshape(1, C)
    Wl2T, Wr2T, b2 = Wl2.T, Wr2.T, bl2.reshape(1, C)
    Wl3T, Wr3T, b3 = Wl3.T, Wr3.T, bl3.reshape(1, C)

    P1 = _segsum16(x16, ed, z16)
    h1 = _tc_layer(P1, P1, x16, Wl1T, b1, Wr1T, split_out=True)

    Wls = jnp.stack([Wl2T, Wl3T])
    bs = jnp.stack([b2, b3])
    Wrs = jnp.stack([Wr2T, Wr3T])

    def step(h, w):
        wl, bb, wr = w
        P = _segsum128(h, ed, z64)
        return _tc_layer(P, P1, h, wl, bb, wr, split_out=True), 0.0

    h3, _ = lax.scan(step, h1, (Wls, bs, Wrs))
    out = jnp.concatenate(
        [h3[c, k] for c in range(NC) for k in range(KP)], axis=1)
    return out[:N]
